# baseline (speedup 1.0000x reference)
import functools

import jax
import jax.numpy as jnp
from jax import lax
from jax.experimental import pallas as pl
from jax.experimental.pallas import tpu as pltpu
from jax.experimental.pallas import tpu_sc as plsc

_NC = 2
_NS = 16
_NW = _NC * _NS
_R = 256
_NBUF = 3
_K = 1


def _ring_copy(src, dst, base, rows, vb, fsem, dsem):
    nb = rows // _R
    fills = [
        pltpu.make_async_copy(
            src.at[pl.ds(base + b * _R, _R)], vb.at[b % _NBUF], fsem.at[b % _NBUF]
        )
        for b in range(nb)
    ]
    drains = [
        pltpu.make_async_copy(
            vb.at[b % _NBUF], dst.at[pl.ds(base + b * _R, _R)], dsem.at[b % _NBUF]
        )
        for b in range(nb)
    ]
    for b in range(min(_NBUF, nb)):
        fills[b].start()
    waited = -1
    for b in range(nb):
        fills[b].wait()
        drains[b].start()
        j = b - _K
        if j >= 0 and j + _NBUF < nb:
            drains[j].wait()
            fills[j + _NBUF].start()
            waited = j
    for b in range(waited + 1, nb):
        drains[b].wait()


def _sc_body(emb, lab, eq, lq, out_eq, out_lq, vb, vl, fsem, dsem, lfsem, ldsem):
    B = emb.shape[0]
    S = eq.shape[0]
    rows_pw = S // _NW
    nw_emb = B // rows_pw
    wid = lax.axis_index("s") * _NC + lax.axis_index("c")
    base = wid * rows_pw

    @pl.when(wid < nw_emb)
    def _():
        _ring_copy(emb, out_eq, base, rows_pw, vb, fsem, dsem)
        cf = pltpu.make_async_copy(lab.at[pl.ds(base, rows_pw)], vl, lfsem)
        cf.start()
        cf.wait()
        cd = pltpu.make_async_copy(vl, out_lq.at[pl.ds(base, rows_pw)], ldsem)
        cd.start()
        cd.wait()

    @pl.when(wid >= nw_emb)
    def _():
        _ring_copy(eq, out_eq, base, rows_pw, vb, fsem, dsem)
        cf = pltpu.make_async_copy(lq.at[pl.ds(base, rows_pw)], vl, lfsem)
        cf.start()
        cf.wait()
        cd = pltpu.make_async_copy(vl, out_lq.at[pl.ds(base, rows_pw)], ldsem)
        cd.start()
        cd.wait()


def kernel(embeddings, labels, embed_queue, label_queue):
    B, D = embeddings.shape
    S = embed_queue.shape[0]
    rows_pw = S // _NW
    mesh = plsc.VectorSubcoreMesh(core_axis_name="c", subcore_axis_name="s")
    sc_call = functools.partial(
        pl.kernel,
        mesh=mesh,
        out_type=[
            jax.ShapeDtypeStruct(embed_queue.shape, embed_queue.dtype),
            jax.ShapeDtypeStruct(label_queue.shape, label_queue.dtype),
        ],
        scratch_types=[
            pltpu.VMEM((_NBUF, _R, D), embed_queue.dtype),
            pltpu.VMEM((rows_pw,), label_queue.dtype),
            pltpu.SemaphoreType.DMA((_NBUF,)),
            pltpu.SemaphoreType.DMA((_NBUF,)),
            pltpu.SemaphoreType.DMA,
            pltpu.SemaphoreType.DMA,
        ],
    )
    out_eq, out_lq = sc_call(_sc_body)(embeddings, labels, embed_queue, label_queue)
    new_ptr = jnp.array([B % S], dtype=jnp.int32)
    return out_eq, out_lq, new_ptr


# speedup vs baseline: 1.1512x; 1.1512x over previous
import functools

import jax
import jax.numpy as jnp
from jax import lax
from jax.experimental import pallas as pl
from jax.experimental.pallas import tpu as pltpu
from jax.experimental.pallas import tpu_sc as plsc

_R = 512
_NBUF = 64
_K = 32

_NC = 2
_NS = 16
_NW = _NC * _NS


def _tc_body(emb, eq, out_eq, vb, fsem, dsem):
    S, D = out_eq.shape
    B = emb.shape[0]
    nb = S // _R
    nb_emb = B // _R

    fills = [
        pltpu.make_async_copy(
            (emb if b < nb_emb else eq).at[pl.ds(b * _R, _R)],
            vb.at[b % _NBUF],
            fsem.at[b % _NBUF],
        )
        for b in range(nb)
    ]
    drains = [
        pltpu.make_async_copy(
            vb.at[b % _NBUF],
            out_eq.at[pl.ds(b * _R, _R)],
            dsem.at[b % _NBUF],
        )
        for b in range(nb)
    ]
    for b in range(min(_NBUF, nb)):
        fills[b].start()
    waited = -1
    for b in range(nb):
        fills[b].wait()
        drains[b].start()
        j = b - _K
        if j >= 0 and j + _NBUF < nb:
            drains[j].wait()
            fills[j + _NBUF].start()
            waited = j
    for b in range(waited + 1, nb):
        drains[b].wait()


def _sc_body(lab, lq, out_lq, vl, fsem, dsem):
    B = lab.shape[0]
    S = lq.shape[0]
    n_pw = S // _NW
    nw_lab = B // n_pw
    wid = lax.axis_index("s") * _NC + lax.axis_index("c")
    base = wid * n_pw

    @pl.when(wid < nw_lab)
    def _():
        cf = pltpu.make_async_copy(lab.at[pl.ds(base, n_pw)], vl, fsem)
        cf.start()
        cf.wait()
        cd = pltpu.make_async_copy(vl, out_lq.at[pl.ds(base, n_pw)], dsem)
        cd.start()
        cd.wait()

    @pl.when(wid >= nw_lab)
    def _():
        cf = pltpu.make_async_copy(lq.at[pl.ds(base, n_pw)], vl, fsem)
        cf.start()
        cf.wait()
        cd = pltpu.make_async_copy(vl, out_lq.at[pl.ds(base, n_pw)], dsem)
        cd.start()
        cd.wait()


def kernel(embeddings, labels, embed_queue, label_queue):
    B, D = embeddings.shape
    S = embed_queue.shape[0]

    mesh = plsc.VectorSubcoreMesh(core_axis_name="c", subcore_axis_name="s")
    sc_call = functools.partial(
        pl.kernel,
        mesh=mesh,
        out_type=jax.ShapeDtypeStruct(label_queue.shape, label_queue.dtype),
        scratch_types=[
            pltpu.VMEM((S // _NW,), label_queue.dtype),
            pltpu.SemaphoreType.DMA,
            pltpu.SemaphoreType.DMA,
        ],
    )
    out_lq = sc_call(_sc_body)(labels, label_queue)

    out_eq = pl.pallas_call(
        _tc_body,
        in_specs=[pl.BlockSpec(memory_space=pl.ANY)] * 2,
        out_specs=pl.BlockSpec(memory_space=pl.ANY),
        out_shape=jax.ShapeDtypeStruct(embed_queue.shape, embed_queue.dtype),
        scratch_shapes=[
            pltpu.VMEM((_NBUF, _R, D), embed_queue.dtype),
            pltpu.SemaphoreType.DMA((_NBUF,)),
            pltpu.SemaphoreType.DMA((_NBUF,)),
        ],
    )(embeddings, embed_queue)

    new_ptr = jnp.array([B % S], dtype=jnp.int32)
    return out_eq, out_lq, new_ptr


# speedup vs baseline: 1.9670x; 1.7086x over previous
import jax
import jax.numpy as jnp
from jax.experimental import pallas as pl
from jax.experimental.pallas import tpu as pltpu

_R = 1024
_NBUF = 32
_K = 16


def _copy_body(emb, lab, eq, lq, out_eq, out_lq, vb, vlab, fsem, dsem, lsem):
    S, D = out_eq.shape
    B = emb.shape[0]
    nb = S // _R
    nb_emb = B // _R

    fills = [
        pltpu.make_async_copy(
            (emb if b < nb_emb else eq).at[pl.ds(b * _R, _R)],
            vb.at[b % _NBUF],
            fsem.at[b % _NBUF],
        )
        for b in range(nb)
    ]
    drains = [
        pltpu.make_async_copy(
            vb.at[b % _NBUF],
            out_eq.at[pl.ds(b * _R, _R)],
            dsem.at[b % _NBUF],
        )
        for b in range(nb)
    ]
    rl = lab.shape[0]
    ltail = lq.shape[0] - rl
    lfill1 = pltpu.make_async_copy(lab, vlab.at[pl.ds(0, rl)], lsem.at[0])
    lfill2 = pltpu.make_async_copy(
        lq.at[pl.ds(rl, ltail)], vlab.at[pl.ds(rl, ltail)], lsem.at[0]
    )
    ldrain = pltpu.make_async_copy(vlab, out_lq, lsem.at[1])

    lfill1.start()
    lfill2.start()
    for b in range(_NBUF):
        fills[b].start()
    lfill1.wait()
    lfill2.wait()
    ldrain.start()
    for b in range(nb):
        fills[b].wait()
        drains[b].start()
        j = b - _K
        if j >= 0 and j + _NBUF < nb:
            drains[j].wait()
            fills[j + _NBUF].start()
    waited = [j for j in range(nb) if j + _NBUF < nb and j <= nb - 1 - _K]
    first_unwaited = (waited[-1] + 1) if waited else 0
    for b in range(first_unwaited, nb):
        drains[b].wait()
    ldrain.wait()


def kernel(embeddings, labels, embed_queue, label_queue):
    B, D = embeddings.shape
    S = embed_queue.shape[0]
    lab2 = labels.reshape(B // 128, 128)
    lq2 = label_queue.reshape(S // 128, 128)
    out_eq, out_lq = pl.pallas_call(
        _copy_body,
        in_specs=[pl.BlockSpec(memory_space=pl.ANY)] * 4,
        out_specs=[pl.BlockSpec(memory_space=pl.ANY)] * 2,
        out_shape=[
            jax.ShapeDtypeStruct(embed_queue.shape, embed_queue.dtype),
            jax.ShapeDtypeStruct(lq2.shape, lq2.dtype),
        ],
        scratch_shapes=[
            pltpu.VMEM((_NBUF, _R, D), embed_queue.dtype),
            pltpu.VMEM((S // 128, 128), label_queue.dtype),
            pltpu.SemaphoreType.DMA((_NBUF,)),
            pltpu.SemaphoreType.DMA((_NBUF,)),
            pltpu.SemaphoreType.DMA((2,)),
        ],
    )(embeddings, lab2, embed_queue, lq2)
    new_ptr = jnp.array([B % S], dtype=jnp.int32)
    return out_eq, out_lq.reshape(S), new_ptr


# speedup vs baseline: 2.0052x; 1.0195x over previous
import jax
import jax.numpy as jnp
from jax.experimental import pallas as pl
from jax.experimental.pallas import tpu as pltpu

_R = 2048
_NBUF = 16
_K = 8


def _copy_body(emb, lab, eq, lq, out_eq, out_lq, vb, vlab, fsem, dsem, lsem):
    S, D = out_eq.shape
    B = emb.shape[0]
    nb = S // _R
    nb_emb = B // _R

    fills = [
        pltpu.make_async_copy(
            (emb if b < nb_emb else eq).at[pl.ds(b * _R, _R)],
            vb.at[b % _NBUF],
            fsem.at[b % _NBUF],
        )
        for b in range(nb)
    ]
    drains = [
        pltpu.make_async_copy(
            vb.at[b % _NBUF],
            out_eq.at[pl.ds(b * _R, _R)],
            dsem.at[b % _NBUF],
        )
        for b in range(nb)
    ]
    rl = lab.shape[0]
    ltail = lq.shape[0] - rl
    lfill1 = pltpu.make_async_copy(lab, vlab.at[pl.ds(0, rl)], lsem.at[0])
    lfill2 = pltpu.make_async_copy(
        lq.at[pl.ds(rl, ltail)], vlab.at[pl.ds(rl, ltail)], lsem.at[0]
    )
    ldrain = pltpu.make_async_copy(vlab, out_lq, lsem.at[1])

    lfill1.start()
    lfill2.start()
    for b in range(_NBUF):
        fills[b].start()
    lfill1.wait()
    lfill2.wait()
    ldrain.start()
    for b in range(nb):
        fills[b].wait()
        drains[b].start()
        j = b - _K
        if j >= 0 and j + _NBUF < nb:
            drains[j].wait()
            fills[j + _NBUF].start()
    waited = [j for j in range(nb) if j + _NBUF < nb and j <= nb - 1 - _K]
    first_unwaited = (waited[-1] + 1) if waited else 0
    for b in range(first_unwaited, nb):
        drains[b].wait()
    ldrain.wait()


def kernel(embeddings, labels, embed_queue, label_queue):
    B, D = embeddings.shape
    S = embed_queue.shape[0]
    lab2 = labels.reshape(B // 128, 128)
    lq2 = label_queue.reshape(S // 128, 128)
    out_eq, out_lq = pl.pallas_call(
        _copy_body,
        in_specs=[pl.BlockSpec(memory_space=pl.ANY)] * 4,
        out_specs=[pl.BlockSpec(memory_space=pl.ANY)] * 2,
        out_shape=[
            jax.ShapeDtypeStruct(embed_queue.shape, embed_queue.dtype),
            jax.ShapeDtypeStruct(lq2.shape, lq2.dtype),
        ],
        scratch_shapes=[
            pltpu.VMEM((_NBUF, _R, D), embed_queue.dtype),
            pltpu.VMEM((S // 128, 128), label_queue.dtype),
            pltpu.SemaphoreType.DMA((_NBUF,)),
            pltpu.SemaphoreType.DMA((_NBUF,)),
            pltpu.SemaphoreType.DMA((2,)),
        ],
    )(embeddings, lab2, embed_queue, lq2)
    new_ptr = jnp.array([B % S], dtype=jnp.int32)
    return out_eq, out_lq.reshape(S), new_ptr


# speedup vs baseline: 2.0071x; 1.0009x over previous
import jax
import jax.numpy as jnp
from jax.experimental import pallas as pl
from jax.experimental.pallas import tpu as pltpu

_R = 4096
_NBUF = 8
_K = 4


def _copy_body(emb, lab, eq, lq, out_eq, out_lq, vb, vlab, fsem, dsem, lsem):
    S, D = out_eq.shape
    B = emb.shape[0]
    nb = S // _R
    nb_emb = B // _R

    fills = [
        pltpu.make_async_copy(
            (emb if b < nb_emb else eq).at[pl.ds(b * _R, _R)],
            vb.at[b % _NBUF],
            fsem.at[b % _NBUF],
        )
        for b in range(nb)
    ]
    drains = [
        pltpu.make_async_copy(
            vb.at[b % _NBUF],
            out_eq.at[pl.ds(b * _R, _R)],
            dsem.at[b % _NBUF],
        )
        for b in range(nb)
    ]
    rl = lab.shape[0]
    ltail = lq.shape[0] - rl
    lfill1 = pltpu.make_async_copy(lab, vlab.at[pl.ds(0, rl)], lsem.at[0])
    lfill2 = pltpu.make_async_copy(
        lq.at[pl.ds(rl, ltail)], vlab.at[pl.ds(rl, ltail)], lsem.at[0]
    )
    ldrain = pltpu.make_async_copy(vlab, out_lq, lsem.at[1])

    lfill1.start()
    lfill2.start()
    for b in range(_NBUF):
        fills[b].start()
    lfill1.wait()
    lfill2.wait()
    ldrain.start()
    for b in range(nb):
        fills[b].wait()
        drains[b].start()
        j = b - _K
        if j >= 0 and j + _NBUF < nb:
            drains[j].wait()
            fills[j + _NBUF].start()
    waited = [j for j in range(nb) if j + _NBUF < nb and j <= nb - 1 - _K]
    first_unwaited = (waited[-1] + 1) if waited else 0
    for b in range(first_unwaited, nb):
        drains[b].wait()
    ldrain.wait()


def kernel(embeddings, labels, embed_queue, label_queue):
    B, D = embeddings.shape
    S = embed_queue.shape[0]
    lab2 = labels.reshape(B // 128, 128)
    lq2 = label_queue.reshape(S // 128, 128)
    out_eq, out_lq = pl.pallas_call(
        _copy_body,
        in_specs=[pl.BlockSpec(memory_space=pl.ANY)] * 4,
        out_specs=[pl.BlockSpec(memory_space=pl.ANY)] * 2,
        out_shape=[
            jax.ShapeDtypeStruct(embed_queue.shape, embed_queue.dtype),
            jax.ShapeDtypeStruct(lq2.shape, lq2.dtype),
        ],
        scratch_shapes=[
            pltpu.VMEM((_NBUF, _R, D), embed_queue.dtype),
            pltpu.VMEM((S // 128, 128), label_queue.dtype),
            pltpu.SemaphoreType.DMA((_NBUF,)),
            pltpu.SemaphoreType.DMA((_NBUF,)),
            pltpu.SemaphoreType.DMA((2,)),
        ],
    )(embeddings, lab2, embed_queue, lq2)
    new_ptr = jnp.array([B % S], dtype=jnp.int32)
    return out_eq, out_lq.reshape(S), new_ptr
